# Initial kernel scaffold; baseline (speedup 1.0000x reference)
import jax
import jax.numpy as jnp
from jax.experimental import pallas as pl


def kernel(gene_idx, disease_idx, gene_feat, disease_feat, gene_table, disease_table, Wg, bg, Wd, bd, W1, b1, W2, b2, Wout, bout):
    raise NotImplementedError("write your pallas kernel here")


# speedup vs baseline: 1.2760x; 1.2760x over previous
import functools

import jax
import jax.numpy as jnp
from jax import lax
from jax.experimental import pallas as pl
from jax.experimental.pallas import tpu as pltpu
from jax.experimental.pallas import tpu_sc as plsc

BATCH = 16384
EMB = 32


def _make_sc_gather(B, D):
    info = plsc.get_sparse_core_info()
    NW = info.num_cores * info.num_subcores
    b_per_w = B // NW
    CH = 128
    n_ch = b_per_w // CH
    nc = info.num_cores
    mesh = plsc.VectorSubcoreMesh(core_axis_name="c", subcore_axis_name="s")

    @functools.partial(
        pl.kernel,
        mesh=mesh,
        out_type=[
            jax.ShapeDtypeStruct((B, D), jnp.float32),
            jax.ShapeDtypeStruct((B, D), jnp.float32),
        ],
        scratch_types=[
            pltpu.VMEM((b_per_w,), jnp.int32),
            pltpu.VMEM((b_per_w,), jnp.int32),
            pltpu.VMEM((b_per_w, D), jnp.float32),
            pltpu.VMEM((b_per_w, D), jnp.float32),
            pltpu.SemaphoreType.DMA,
        ],
        compiler_params=pltpu.CompilerParams(use_tc_tiling_on_sc=False),
    )
    def sc_gather(gt_hbm, dt_hbm, gi_hbm, di_hbm, ge_out, de_out,
                  gi_v, di_v, gr_v, dr_v, sem):
        wid = lax.axis_index("s") * nc + lax.axis_index("c")
        base = wid * b_per_w
        pltpu.sync_copy(gi_hbm.at[pl.ds(base, b_per_w)], gi_v)
        pltpu.sync_copy(di_hbm.at[pl.ds(base, b_per_w)], di_v)
        copies = []
        for j in range(n_ch):
            sl = pl.ds(j * CH, CH)
            copies.append(pltpu.async_copy(gt_hbm.at[gi_v.at[sl]], gr_v.at[sl], sem))
            copies.append(pltpu.async_copy(dt_hbm.at[di_v.at[sl]], dr_v.at[sl], sem))
        for c in copies:
            c.wait()
        pltpu.sync_copy(gr_v, ge_out.at[pl.ds(base, b_per_w)])
        pltpu.sync_copy(dr_v, de_out.at[pl.ds(base, b_per_w)])

    return sc_gather


def _mlp_body(ge_ref, de_ref, gf_ref, df_ref,
              WgT_ref, WdT_ref, W1aT_ref, W1bT_ref, W1cT_ref, W1dT_ref,
              W2T_ref, WoutT_ref, bg_ref, bd_ref, b1_ref, b2_ref, bout_ref,
              out_ref):
    f32 = jnp.float32
    dot = functools.partial(jax.lax.dot_general,
                            dimension_numbers=(((1,), (0,)), ((), ())),
                            preferred_element_type=f32)
    sg = dot(gf_ref[...], WgT_ref[...]) + bg_ref[...]
    sd = dot(df_ref[...], WdT_ref[...]) + bd_ref[...]
    h1 = (dot(ge_ref[...], W1aT_ref[...]) + dot(de_ref[...], W1bT_ref[...])
          + dot(sg, W1cT_ref[...]) + dot(sd, W1dT_ref[...]) + b1_ref[...])
    h1 = jnp.maximum(h1, 0.0)
    h2 = jnp.maximum(dot(h1, W2T_ref[...]) + b2_ref[...], 0.0)
    out_ref[...] = dot(h2, WoutT_ref[...]) + bout_ref[...]


def _tc_mlp(ge, de, gf, df, WgT, WdT, W1aT, W1bT, W1cT, W1dT, W2T, WoutT,
            bg2, bd2, b12, b22, bout2):
    B = ge.shape[0]
    BLK = 2048
    grid = (B // BLK,)

    def row_spec(width):
        return pl.BlockSpec((BLK, width), lambda i: (i, 0))

    def full_spec(a):
        return pl.BlockSpec(a.shape, lambda i: (0, 0))

    return pl.pallas_call(
        _mlp_body,
        grid=grid,
        in_specs=[
            row_spec(EMB), row_spec(EMB), row_spec(64), row_spec(64),
            full_spec(WgT), full_spec(WdT), full_spec(W1aT), full_spec(W1bT),
            full_spec(W1cT), full_spec(W1dT), full_spec(W2T), full_spec(WoutT),
            full_spec(bg2), full_spec(bd2), full_spec(b12), full_spec(b22),
            full_spec(bout2),
        ],
        out_specs=pl.BlockSpec((BLK, 1), lambda i: (i, 0)),
        out_shape=jax.ShapeDtypeStruct((B, 1), jnp.float32),
    )(ge, de, gf, df, WgT, WdT, W1aT, W1bT, W1cT, W1dT, W2T, WoutT,
      bg2, bd2, b12, b22, bout2)


def kernel(gene_idx, disease_idx, gene_feat, disease_feat, gene_table,
           disease_table, Wg, bg, Wd, bd, W1, b1, W2, b2, Wout, bout):
    B = gene_idx.shape[0]
    sc_gather = _make_sc_gather(B, EMB)
    ge, de = sc_gather(gene_table, disease_table,
                       gene_idx.astype(jnp.int32), disease_idx.astype(jnp.int32))

    WgT = Wg.T
    WdT = Wd.T
    W1aT = W1[:, 0 * EMB:1 * EMB].T
    W1bT = W1[:, 1 * EMB:2 * EMB].T
    W1cT = W1[:, 2 * EMB:3 * EMB].T
    W1dT = W1[:, 3 * EMB:4 * EMB].T
    W2T = W2.T
    WoutT = Wout.T
    bg2 = bg.reshape(1, -1)
    bd2 = bd.reshape(1, -1)
    b12 = b1.reshape(1, -1)
    b22 = b2.reshape(1, -1)
    bout2 = bout.reshape(1, -1)

    out = _tc_mlp(ge, de, gene_feat, disease_feat, WgT, WdT,
                  W1aT, W1bT, W1cT, W1dT, W2T, WoutT,
                  bg2, bd2, b12, b22, bout2)
    return jnp.squeeze(out, axis=-1)


# speedup vs baseline: 1.3093x; 1.0261x over previous
import functools

import jax
import jax.numpy as jnp
from jax import lax
from jax.experimental import pallas as pl
from jax.experimental.pallas import tpu as pltpu
from jax.experimental.pallas import tpu_sc as plsc

EMB = 32


def _make_sc_gather(B, D):
    info = plsc.get_sparse_core_info()
    NW = info.num_cores * info.num_subcores
    b_per_w = B // NW
    CH = 128
    n_ch = b_per_w // CH
    nc = info.num_cores
    mesh = plsc.VectorSubcoreMesh(core_axis_name="c", subcore_axis_name="s")

    @functools.partial(
        pl.kernel,
        mesh=mesh,
        out_type=jax.ShapeDtypeStruct((B, 128), jnp.float32),
        scratch_types=[
            pltpu.VMEM((b_per_w,), jnp.int32),
            pltpu.VMEM((b_per_w,), jnp.int32),
            pltpu.VMEM((b_per_w, D), jnp.float32),
            pltpu.VMEM((b_per_w, D), jnp.float32),
            pltpu.SemaphoreType.DMA,
        ],
        compiler_params=pltpu.CompilerParams(use_tc_tiling_on_sc=False),
    )
    def sc_gather(gt_hbm, dt_hbm, gi_hbm, di_hbm, x_out,
                  gi_v, di_v, gr_v, dr_v, sem):
        wid = lax.axis_index("s") * nc + lax.axis_index("c")
        base = wid * b_per_w
        pltpu.sync_copy(gi_hbm.at[pl.ds(base, b_per_w)], gi_v)
        pltpu.sync_copy(di_hbm.at[pl.ds(base, b_per_w)], di_v)
        copies = []
        for j in range(n_ch):
            sl = pl.ds(j * CH, CH)
            copies.append(pltpu.async_copy(gt_hbm.at[gi_v.at[sl]], gr_v.at[sl], sem))
            copies.append(pltpu.async_copy(dt_hbm.at[di_v.at[sl]], dr_v.at[sl], sem))
        for c in copies:
            c.wait()
        rows = pl.ds(base, b_per_w)
        pltpu.sync_copy(gr_v, x_out.at[rows, pl.ds(0, D)])
        pltpu.sync_copy(dr_v, x_out.at[rows, pl.ds(D, D)])

    return sc_gather


def _mlp_body(x_ref, gf_ref, df_ref,
              WgT_ref, WdT_ref, W1abT_ref, W1cT_ref, W1dT_ref, W2T_ref,
              bg_ref, bd_ref, b1_ref, b2_ref, wout_ref, bout_ref, out_ref):
    dot = functools.partial(jax.lax.dot_general,
                            dimension_numbers=(((1,), (0,)), ((), ())),
                            preferred_element_type=jnp.float32)
    sg = dot(gf_ref[...], WgT_ref[...]) + bg_ref[...]
    sd = dot(df_ref[...], WdT_ref[...]) + bd_ref[...]
    h1 = (dot(x_ref[:, 0:2 * EMB], W1abT_ref[...])
          + dot(sg, W1cT_ref[...])
          + dot(sd, W1dT_ref[...])
          + b1_ref[...])
    h1 = jnp.maximum(h1, 0.0)
    h2 = jnp.maximum(dot(h1, W2T_ref[...]) + b2_ref[...], 0.0)
    h2b = h2.astype(jnp.bfloat16).astype(jnp.float32)
    wb = wout_ref[...].astype(jnp.bfloat16).astype(jnp.float32)
    out_ref[...] = jnp.sum(h2b * wb, axis=1) + bout_ref[0, 0]


def _tc_mlp(x, gf, df, WgT, WdT, W1abT, W1cT, W1dT, W2T,
            bg2, bd2, b12, b22, wout2, bout2):
    B = x.shape[0]
    BLK = 2048
    grid = (B // BLK,)

    def full_spec(a):
        return pl.BlockSpec(a.shape, lambda i: (0, 0))

    return pl.pallas_call(
        _mlp_body,
        grid=grid,
        in_specs=[
            pl.BlockSpec((BLK, 128), lambda i: (i, 0)),
            pl.BlockSpec((BLK, 64), lambda i: (i, 0)),
            pl.BlockSpec((BLK, 64), lambda i: (i, 0)),
            full_spec(WgT), full_spec(WdT), full_spec(W1abT),
            full_spec(W1cT), full_spec(W1dT), full_spec(W2T),
            full_spec(bg2), full_spec(bd2), full_spec(b12), full_spec(b22),
            full_spec(wout2), full_spec(bout2),
        ],
        out_specs=pl.BlockSpec((BLK,), lambda i: (i,)),
        out_shape=jax.ShapeDtypeStruct((B,), jnp.float32),
    )(x, gf, df, WgT, WdT, W1abT, W1cT, W1dT, W2T,
      bg2, bd2, b12, b22, wout2, bout2)


def kernel(gene_idx, disease_idx, gene_feat, disease_feat, gene_table,
           disease_table, Wg, bg, Wd, bd, W1, b1, W2, b2, Wout, bout):
    B = gene_idx.shape[0]
    sc_gather = _make_sc_gather(B, EMB)
    x = sc_gather(gene_table, disease_table,
                  gene_idx.astype(jnp.int32), disease_idx.astype(jnp.int32))

    W1abT = W1[:, 0:2 * EMB].T
    W1cT = W1[:, 2 * EMB:3 * EMB].T
    W1dT = W1[:, 3 * EMB:4 * EMB].T
    W2T = W2.T
    bg2 = bg.reshape(1, -1)
    bd2 = bd.reshape(1, -1)
    b12 = b1.reshape(1, -1)
    b22 = b2.reshape(1, -1)
    wout2 = Wout.reshape(1, -1)
    bout2 = bout.reshape(1, 1)

    return _tc_mlp(x, gene_feat, disease_feat, Wg.T, Wd.T,
                   W1abT, W1cT, W1dT, W2T, bg2, bd2, b12, b22, wout2, bout2)
